# baseline (speedup 1.0000x reference)
import functools

import jax
import jax.numpy as jnp
from jax import lax
from jax.experimental import pallas as pl
from jax.experimental.pallas import tpu as pltpu
from jax.experimental.pallas import tpu_sc as plsc

_NC = 2
_NS = 16
_NW = _NC * _NS
_L = 16

_CHUNK = 128


def _tt_kernel(n_total, ntok, embed, nbits_words,
               x_hbm, tok_hbm, delta_hbm, w_hbm, zeros_hbm, out_hbm,
               idx_v, rows_v, tok_v, delta_v, bitmap_v, sem, zsem):
    wid = lax.axis_index("s") * _NC + lax.axis_index("c")
    n_per_w = n_total // _NW
    n_chunks = n_per_w // _CHUNK

    zcopy = pltpu.async_copy(zeros_hbm, bitmap_v, zsem)
    pltpu.sync_copy(tok_hbm, tok_v)
    pltpu.sync_copy(delta_hbm, delta_v)
    zcopy.wait()

    lane = lax.iota(jnp.int32, _L)

    def set_bit(t, _):
        tok = tok_v[pl.ds((t // _L) * _L, _L)]
        w = lax.shift_right_logical(tok, 5)
        b = lax.shift_left(jnp.int32(1), jnp.bitwise_and(tok, 31))
        one = lane == (t % _L)
        old = plsc.load_gather(bitmap_v, [w], mask=one)
        plsc.store_scatter(bitmap_v, [w], old | b, mask=one)
        return 0

    lax.fori_loop(0, ntok, set_bit, 0)

    def do_chunk(c, _):
        base = wid * n_per_w + c * _CHUNK
        pltpu.sync_copy(x_hbm.at[pl.ds(base, _CHUNK)], idx_v)
        pltpu.async_copy(w_hbm.at[idx_v], rows_v, sem).wait()

        def check_vec(v, _):
            xv = idx_v[pl.ds(v * _L, _L)]
            w = lax.shift_right_logical(xv, 5)
            b = lax.shift_left(jnp.int32(1), jnp.bitwise_and(xv, 31))
            hit = (plsc.load_gather(bitmap_v, [w]) & b) != 0

            @pl.when(jnp.any(hit))
            def _fix():
                def find_vec(tt, jv):
                    tokvec = tok_v[pl.ds(tt * _L, _L)]

                    def find_lane(e, jv):
                        tval = tokvec.at[jnp.full((_L,), e, jnp.int32)].get(
                            mode="promise_in_bounds")
                        return jnp.where(xv == tval, tt * _L + e, jv)

                    return lax.fori_loop(0, _L, find_lane, jv)

                jv = lax.fori_loop(0, ntok // _L, find_vec,
                                   jnp.zeros(_L, jnp.int32))
                row = v * _L + lane

                def copy_col(col, _):
                    cv = jnp.full((_L,), col, jnp.int32)
                    dval = plsc.load_gather(delta_v, [jv, cv], mask=hit)
                    plsc.store_scatter(rows_v, [row, cv], dval, mask=hit)
                    return 0

                lax.fori_loop(0, embed, copy_col, 0)

            return 0

        lax.fori_loop(0, _CHUNK // _L, check_vec, 0)
        pltpu.sync_copy(rows_v, out_hbm.at[pl.ds(base, _CHUNK)])
        return 0

    lax.fori_loop(0, n_chunks, do_chunk, 0)


def kernel(x, W, token_indices, delta):
    b, l = x.shape
    vocab, embed = W.shape
    ntok = token_indices.shape[0]
    n_total = b * l
    nbits_words = (vocab + 31) // 32

    xf = x.reshape(n_total).astype(jnp.int32)
    tok = token_indices.astype(jnp.int32)
    zeros = jnp.zeros((nbits_words,), jnp.int32)

    mesh = plsc.VectorSubcoreMesh(
        core_axis_name="c", subcore_axis_name="s",
        num_cores=_NC, num_subcores=_NS)

    body = functools.partial(_tt_kernel, n_total, ntok, embed, nbits_words)
    out = pl.kernel(
        body,
        out_type=jax.ShapeDtypeStruct((n_total, embed), jnp.float32),
        mesh=mesh,
        compiler_params=pltpu.CompilerParams(
            needs_layout_passes=False, use_tc_tiling_on_sc=False),
        scratch_types=[
            pltpu.VMEM((_CHUNK,), jnp.int32),
            pltpu.VMEM((_CHUNK, embed), jnp.float32),
            pltpu.VMEM((ntok,), jnp.int32),
            pltpu.VMEM((ntok, embed), jnp.float32),
            pltpu.VMEM((nbits_words,), jnp.int32),
            pltpu.SemaphoreType.DMA,
            pltpu.SemaphoreType.DMA,
        ],
    )(xf, tok, delta, W, zeros)
    return out.reshape(b, l, embed)
